# baseline (speedup 1.0000x reference)
import functools

import jax
import jax.numpy as jnp
from jax import lax
from jax.experimental import pallas as pl
from jax.experimental.pallas import tpu as pltpu
from jax.experimental.pallas import tpu_sc as plsc

_B, _N, _H = 128, 128, 32
_MAX_DIST, _MAX_BOND = 37, 4
_NCLS = _MAX_DIST * _MAX_BOND
_P = _N * _N
_NC, _NS = 2, 16
_NW = _NC * _NS
_B_PER_W = _B // _NW
_NPACK = _NCLS * _H // 2
_CHUNK = 512
_GROUPS = _CHUNK // 16
_PAIRS_PER_B = _P // (2 * _CHUNK)
_NPAIR = _B_PER_W * _PAIRS_PER_B


def _build_table(ew_v, dw_v, tab_v, rep_v):
    iota16 = lax.iota(jnp.int32, 16)
    iota_e = iota16 * 2
    ew_e = [plsc.load_gather(ew_v, [a * _H + iota_e])
            for a in range(1, _MAX_BOND)]
    ew_o = [plsc.load_gather(ew_v, [a * _H + iota_e + 1])
            for a in range(1, _MAX_BOND)]

    def body(d, carry):
        d_e = plsc.load_gather(dw_v, [d * _H + iota_e])
        d_o = plsc.load_gather(dw_v, [d * _H + iota_e + 1])
        for a in range(_MAX_BOND):
            if a == 0:
                v_e, v_o = d_e, d_o
            else:
                v_e, v_o = d_e + ew_e[a - 1], d_o + ew_o[a - 1]
            packed = plsc.pack(v_e, v_o, format=plsc.PackFormat.INTERLEAVED)
            word = plsc.bitcast(packed, jnp.int32)
            plsc.store_scatter(
                tab_v, [iota16 * _NCLS + (d * _MAX_BOND + a)], word)
        return carry

    lax.fori_loop(0, _MAX_DIST, body, 0)

    def rep_body(xg, carry):
        w16 = tab_v[pl.ds(xg * 16, 16)]
        for l in range(16):
            rep_v[pl.ds(xg * 256 + l * 16, 16)] = jnp.full(
                (16,), w16[l], jnp.int32)
        return carry

    lax.fori_loop(0, _NPACK // 16, rep_body, 0)


@functools.partial(
    pl.kernel,
    mesh=plsc.VectorSubcoreMesh(core_axis_name="c", subcore_axis_name="s"),
    compiler_params=pltpu.CompilerParams(needs_layout_passes=False),
    out_type=jax.ShapeDtypeStruct((_B, _H, _P), jnp.float32),
    scratch_types=[
        pltpu.VMEM((_MAX_BOND * _H,), jnp.float32),
        pltpu.VMEM((_MAX_DIST * _H,), jnp.float32),
        pltpu.VMEM((_NPACK,), jnp.int32),
        pltpu.VMEM((_NPACK * 16,), jnp.int32),
        pltpu.VMEM((_P,), jnp.int32),
        pltpu.VMEM((_P,), jnp.int32),
        pltpu.VMEM((_H, _CHUNK), jnp.float32),
        pltpu.VMEM((_H, _CHUNK), jnp.float32),
        pltpu.SemaphoreType.DMA,
        pltpu.SemaphoreType.DMA,
    ],
)
def _sc_bias(ew_hbm, dw_hbm, dist_hbm, adj_hbm, out_hbm,
             ew_v, dw_v, tab_v, rep_v, dist_v, adj_v, out_a, out_b,
             sem_a, sem_b):
    wid = lax.axis_index("s") * _NC + lax.axis_index("c")

    pltpu.sync_copy(ew_hbm, ew_v)
    pltpu.sync_copy(dw_hbm, dw_v)
    _build_table(ew_v, dw_v, tab_v, rep_v)
    iota16 = lax.iota(jnp.int32, 16)

    def fill(out_v, off):
        def group_body(g, carry):
            o = off + g * 16
            d = dist_v[pl.ds(o, 16)]
            a = adj_v[pl.ds(o, 16)]
            cidx = (d * _MAX_BOND + a) * 16 + iota16
            vals = []
            for h2 in range(_H // 2):
                idx = cidx + h2 * (_NCLS * 16) if h2 else cidx
                word = plsc.load_gather(rep_v, [idx])
                v_e, v_o = plsc.unpack(
                    plsc.bitcast(word, jnp.bfloat16),
                    format=plsc.PackFormat.INTERLEAVED,
                    preferred_element_type=jnp.float32)
                vals.append(v_e)
                vals.append(v_o)
            for h2 in range(_H // 2):
                out_v[2 * h2, pl.ds(g * 16, 16)] = vals[2 * h2]
                out_v[2 * h2 + 1, pl.ds(g * 16, 16)] = vals[2 * h2 + 1]
            return carry

        lax.fori_loop(0, _GROUPS, group_body, 0)

    def drain(out_v, sem):
        pltpu.make_async_copy(
            out_v, out_hbm.at[0, :, pl.ds(0, _CHUNK)], sem).wait()

    def pair_body(p, carry):
        b = wid * _B_PER_W + p // _PAIRS_PER_B
        base = (p % _PAIRS_PER_B) * (2 * _CHUNK)

        @pl.when(p % _PAIRS_PER_B == 0)
        def _():
            pltpu.sync_copy(dist_hbm.at[b], dist_v)
            pltpu.sync_copy(adj_hbm.at[b], adj_v)

        @pl.when(p > 0)
        def _():
            drain(out_a, sem_a)

        fill(out_a, base)
        pltpu.async_copy(out_a, out_hbm.at[b, :, pl.ds(base, _CHUNK)], sem_a)

        @pl.when(p > 0)
        def _():
            drain(out_b, sem_b)

        fill(out_b, base + _CHUNK)
        pltpu.async_copy(
            out_b, out_hbm.at[b, :, pl.ds(base + _CHUNK, _CHUNK)], sem_b)
        return carry

    lax.fori_loop(0, _NPAIR, pair_body, 0)
    drain(out_a, sem_a)
    drain(out_b, sem_b)


def kernel(distance, adj, edge_weight, distance_weight):
    dist2 = distance.reshape(_B, _P).astype(jnp.int32)
    adj2 = adj.reshape(_B, _P).astype(jnp.int32)
    out = _sc_bias(edge_weight.reshape(-1), distance_weight.reshape(-1),
                   dist2, adj2)
    return out.reshape(_B, _H, _N, _N)


# speedup vs baseline: 2.1467x; 2.1467x over previous
import functools

import jax
import jax.numpy as jnp
from jax import lax
from jax.experimental import pallas as pl
from jax.experimental.pallas import tpu as pltpu
from jax.experimental.pallas import tpu_sc as plsc

_B, _N, _H = 128, 128, 32
_MAX_DIST, _MAX_BOND = 37, 4
_NCLS = _MAX_DIST * _MAX_BOND
_NC, _NS = 2, 16
_NW = _NC * _NS
_B_PER_W = _B // _NW
_NPACK = _NCLS * _H // 2
_ROWS = 8
_GROUPS = _ROWS * _N // 16
_PAIRS_PER_B = _N // (2 * _ROWS)
_NPAIR = _B_PER_W * _PAIRS_PER_B


def _build_table(ew_v, dw_v, tab_v):
    iota16 = lax.iota(jnp.int32, 16)
    iota_e = iota16 * 2
    ew_e = [plsc.load_gather(ew_v, [a * _H + iota_e])
            for a in range(1, _MAX_BOND)]
    ew_o = [plsc.load_gather(ew_v, [a * _H + iota_e + 1])
            for a in range(1, _MAX_BOND)]

    def body(d, carry):
        d_e = plsc.load_gather(dw_v, [d * _H + iota_e])
        d_o = plsc.load_gather(dw_v, [d * _H + iota_e + 1])
        for a in range(_MAX_BOND):
            if a == 0:
                v_e, v_o = d_e, d_o
            else:
                v_e, v_o = d_e + ew_e[a - 1], d_o + ew_o[a - 1]
            packed = plsc.pack(v_e, v_o, format=plsc.PackFormat.INTERLEAVED)
            word = plsc.bitcast(packed, jnp.int32)
            plsc.store_scatter(
                tab_v, [iota16 * _NCLS + (d * _MAX_BOND + a)], word)
        return carry

    lax.fori_loop(0, _MAX_DIST, body, 0)


@functools.partial(
    pl.kernel,
    mesh=plsc.VectorSubcoreMesh(core_axis_name="c", subcore_axis_name="s"),
    compiler_params=pltpu.CompilerParams(needs_layout_passes=False),
    out_type=jax.ShapeDtypeStruct((_B, _H, _N, _N), jnp.float32),
    scratch_types=[
        pltpu.VMEM((_MAX_BOND * _H,), jnp.float32),
        pltpu.VMEM((_MAX_DIST * _H,), jnp.float32),
        pltpu.VMEM((_NPACK,), jnp.int32),
        pltpu.VMEM((_N, _N), jnp.int32),
        pltpu.VMEM((_N, _N), jnp.int32),
        pltpu.VMEM((_H, _ROWS, _N), jnp.float32),
        pltpu.VMEM((_H, _ROWS, _N), jnp.float32),
        pltpu.SemaphoreType.DMA,
        pltpu.SemaphoreType.DMA,
    ],
)
def _sc_bias(ew_hbm, dw_hbm, dist_hbm, adj_hbm, out_hbm,
             ew_v, dw_v, tab_v, dist_v, adj_v, out_a, out_b,
             sem_a, sem_b):
    wid = lax.axis_index("s") * _NC + lax.axis_index("c")

    pltpu.sync_copy(ew_hbm, ew_v)
    pltpu.sync_copy(dw_hbm, dw_v)
    _build_table(ew_v, dw_v, tab_v)

    def fill(out_v, row0):
        def group_body(g, carry):
            rr = g >> 3
            c0 = (g & 7) << 4
            d = dist_v[row0 + rr, pl.ds(c0, 16)]
            a = adj_v[row0 + rr, pl.ds(c0, 16)]
            cidx = d * _MAX_BOND + a
            vals = []
            for h2 in range(_H // 2):
                idx = cidx + h2 * _NCLS if h2 else cidx
                word = plsc.load_gather(tab_v, [idx])
                v_e, v_o = plsc.unpack(
                    plsc.bitcast(word, jnp.bfloat16),
                    format=plsc.PackFormat.INTERLEAVED,
                    preferred_element_type=jnp.float32)
                vals.append(v_e)
                vals.append(v_o)
            for h in range(_H):
                out_v[h, rr, pl.ds(c0, 16)] = vals[h]
            return carry

        lax.fori_loop(0, _GROUPS, group_body, 0)

    def drain(out_v, sem):
        pltpu.make_async_copy(
            out_v, out_hbm.at[0, :, pl.ds(0, _ROWS), :], sem).wait()

    def pair_body(p, carry):
        b = wid * _B_PER_W + p // _PAIRS_PER_B
        row0 = (p % _PAIRS_PER_B) * (2 * _ROWS)

        @pl.when(p % _PAIRS_PER_B == 0)
        def _():
            pltpu.sync_copy(dist_hbm.at[b], dist_v)
            pltpu.sync_copy(adj_hbm.at[b], adj_v)

        @pl.when(p > 0)
        def _():
            drain(out_a, sem_a)

        fill(out_a, row0)
        pltpu.async_copy(
            out_a, out_hbm.at[b, :, pl.ds(row0, _ROWS), :], sem_a)

        @pl.when(p > 0)
        def _():
            drain(out_b, sem_b)

        fill(out_b, row0 + _ROWS)
        pltpu.async_copy(
            out_b, out_hbm.at[b, :, pl.ds(row0 + _ROWS, _ROWS), :], sem_b)
        return carry

    lax.fori_loop(0, _NPAIR, pair_body, 0)
    drain(out_a, sem_a)
    drain(out_b, sem_b)


def kernel(distance, adj, edge_weight, distance_weight):
    out = _sc_bias(edge_weight.reshape(-1), distance_weight.reshape(-1),
                   distance.astype(jnp.int32), adj)
    return out


# speedup vs baseline: 2.2613x; 1.0534x over previous
import functools

import jax
import jax.numpy as jnp
from jax import lax
from jax.experimental import pallas as pl
from jax.experimental.pallas import tpu as pltpu
from jax.experimental.pallas import tpu_sc as plsc

_B, _N, _H = 128, 128, 32
_MAX_DIST, _MAX_BOND = 37, 4
_NCLS = _MAX_DIST * _MAX_BOND
_NC, _NS = 2, 16
_NW = _NC * _NS
_B_PER_W = _B // _NW
_NPACK = _NCLS * _H // 2
_ROWS = 4
_GROUPS = _ROWS * _N // 16
_PAIRS_PER_B = _N // (2 * _ROWS)
_NPAIR = _B_PER_W * _PAIRS_PER_B


def _build_table(ew_v, dw_v, tab_v):
    iota16 = lax.iota(jnp.int32, 16)
    iota_e = iota16 * 2
    ew_e = [plsc.load_gather(ew_v, [a * _H + iota_e])
            for a in range(1, _MAX_BOND)]
    ew_o = [plsc.load_gather(ew_v, [a * _H + iota_e + 1])
            for a in range(1, _MAX_BOND)]

    def body(d, carry):
        d_e = plsc.load_gather(dw_v, [d * _H + iota_e])
        d_o = plsc.load_gather(dw_v, [d * _H + iota_e + 1])
        for a in range(_MAX_BOND):
            if a == 0:
                v_e, v_o = d_e, d_o
            else:
                v_e, v_o = d_e + ew_e[a - 1], d_o + ew_o[a - 1]
            packed = plsc.pack(v_e, v_o, format=plsc.PackFormat.INTERLEAVED)
            word = plsc.bitcast(packed, jnp.int32)
            plsc.store_scatter(
                tab_v, [iota16 * _NCLS + (d * _MAX_BOND + a)], word)
        return carry

    lax.fori_loop(0, _MAX_DIST, body, 0)


@functools.partial(
    pl.kernel,
    mesh=plsc.VectorSubcoreMesh(core_axis_name="c", subcore_axis_name="s"),
    compiler_params=pltpu.CompilerParams(needs_layout_passes=False),
    out_type=jax.ShapeDtypeStruct((_B, _H, _N, _N), jnp.float32),
    scratch_types=[
        pltpu.VMEM((_MAX_BOND * _H,), jnp.float32),
        pltpu.VMEM((_MAX_DIST * _H,), jnp.float32),
        pltpu.VMEM((_NPACK,), jnp.int32),
        pltpu.VMEM((_N, _N), jnp.int32),
        pltpu.VMEM((_N, _N), jnp.int32),
        pltpu.VMEM((_N, _N), jnp.int32),
        pltpu.VMEM((_N, _N), jnp.int32),
        pltpu.VMEM((_H, _ROWS, _N), jnp.float32),
        pltpu.VMEM((_H, _ROWS, _N), jnp.float32),
        pltpu.SemaphoreType.DMA,
        pltpu.SemaphoreType.DMA,
        pltpu.SemaphoreType.DMA,
    ],
)
def _sc_bias(ew_hbm, dw_hbm, dist_hbm, adj_hbm, out_hbm,
             ew_v, dw_v, tab_v, dist_0, adj_0, dist_1, adj_1, out_a, out_b,
             sem_a, sem_b, sem_in):
    wid = lax.axis_index("s") * _NC + lax.axis_index("c")

    pltpu.sync_copy(ew_hbm, ew_v)
    pltpu.sync_copy(dw_hbm, dw_v)
    _build_table(ew_v, dw_v, tab_v)

    def fill(out_v, row0, dist_v, adj_v):
        def group_body(g, carry):
            rr = g >> 3
            c0 = (g & 7) << 4
            d = dist_v[row0 + rr, pl.ds(c0, 16)]
            a = adj_v[row0 + rr, pl.ds(c0, 16)]
            cidx = d * _MAX_BOND + a
            vals = []
            for h2 in range(_H // 2):
                idx = cidx + h2 * _NCLS if h2 else cidx
                word = plsc.load_gather(tab_v, [idx])
                v_e, v_o = plsc.unpack(
                    plsc.bitcast(word, jnp.bfloat16),
                    format=plsc.PackFormat.INTERLEAVED,
                    preferred_element_type=jnp.float32)
                vals.append(v_e)
                vals.append(v_o)
            for h in range(_H):
                out_v[h, rr, pl.ds(c0, 16)] = vals[h]
            return carry

        lax.fori_loop(0, _GROUPS, group_body, 0)

    def drain(out_v, sem):
        pltpu.make_async_copy(
            out_v, out_hbm.at[0, :, pl.ds(0, _ROWS), :], sem).wait()

    def drain_plane(hbm, plane_v):
        pltpu.make_async_copy(hbm.at[0], plane_v, sem_in).wait()

    b0 = wid * _B_PER_W
    pltpu.async_copy(dist_hbm.at[b0], dist_0, sem_in)
    pltpu.async_copy(adj_hbm.at[b0], adj_0, sem_in)

    for bi in range(_B_PER_W):
        b = b0 + bi
        dist_v, adj_v = (dist_0, adj_0) if bi % 2 == 0 else (dist_1, adj_1)
        drain_plane(dist_hbm, dist_v)
        drain_plane(adj_hbm, adj_v)
        if bi + 1 < _B_PER_W:
            nd, na = (dist_1, adj_1) if bi % 2 == 0 else (dist_0, adj_0)
            pltpu.async_copy(dist_hbm.at[b + 1], nd, sem_in)
            pltpu.async_copy(adj_hbm.at[b + 1], na, sem_in)

        def pair_body(p, carry):
            row0 = p * (2 * _ROWS)

            if bi == 0:
                @pl.when(p > 0)
                def _():
                    drain(out_a, sem_a)
            else:
                drain(out_a, sem_a)

            fill(out_a, row0, dist_v, adj_v)
            pltpu.async_copy(
                out_a, out_hbm.at[b, :, pl.ds(row0, _ROWS), :], sem_a)

            if bi == 0:
                @pl.when(p > 0)
                def _():
                    drain(out_b, sem_b)
            else:
                drain(out_b, sem_b)

            fill(out_b, row0 + _ROWS, dist_v, adj_v)
            pltpu.async_copy(
                out_b, out_hbm.at[b, :, pl.ds(row0 + _ROWS, _ROWS), :], sem_b)
            return carry

        lax.fori_loop(0, _PAIRS_PER_B, pair_body, 0)

    drain(out_a, sem_a)
    drain(out_b, sem_b)


def kernel(distance, adj, edge_weight, distance_weight):
    out = _sc_bias(edge_weight.reshape(-1), distance_weight.reshape(-1),
                   distance.astype(jnp.int32), adj)
    return out
